# baseline (speedup 1.0000x reference)
import functools

import jax
import jax.numpy as jnp
from jax import lax
from jax.experimental import pallas as pl
from jax.experimental.pallas import tpu as pltpu
from jax.experimental.pallas import tpu_sc as plsc

QUANT_TEMP = 0.3
TILE = 2048
N_EXPERTS = 7
L = 16


def _proj_body(x_ref, wt_ref, a_ref, rtc_ref, q_ref, l_ref):
    xt = x_ref[...]
    z = jax.lax.dot_general(
        xt, wt_ref[...], (((1,), (0,)), ((), ())),
        preferred_element_type=jnp.float32)
    qt = jnp.tanh(z.T / QUANT_TEMP)
    q_ref[...] = qt
    dott = jax.lax.dot_general(
        a_ref[...], qt, (((1,), (0,)), ((), ())),
        preferred_element_type=jnp.float32)
    hamming = (6.0 - dott) / 2.0
    l_ref[...] = -hamming / rtc_ref[...]


def _make_sc_routing(n):
    info = plsc.get_sparse_core_info()
    nw = info.num_cores * info.num_subcores
    chunk = n // nw
    mesh = plsc.VectorSubcoreMesh(core_axis_name="c", subcore_axis_name="s")

    @functools.partial(
        pl.kernel, mesh=mesh,
        out_type=jax.ShapeDtypeStruct((N_EXPERTS, n), jnp.float32),
        scratch_types=[
            pltpu.VMEM((N_EXPERTS, chunk), jnp.float32),
            pltpu.VMEM((N_EXPERTS, chunk), jnp.float32),
        ],
    )
    def sc_route(l_hbm, ew_hbm, lv, ewv):
        wid = lax.axis_index("s") * info.num_cores + lax.axis_index("c")
        base = wid * chunk
        pltpu.sync_copy(l_hbm.at[:, pl.ds(base, chunk)], lv)
        neg_inf = jnp.full((L,), -jnp.inf, jnp.float32)
        zero = jnp.zeros((L,), jnp.float32)
        seven = jnp.full((L,), N_EXPERTS, jnp.int32)
        for i in range(chunk // L):
            sl = pl.ds(i * L, L)
            ls = [lv[e, sl] for e in range(N_EXPERTS)]
            m1 = ls[0]
            for e in range(1, N_EXPERTS):
                m1 = jnp.maximum(m1, ls[e])
            i1 = seven
            for e in range(N_EXPERTS - 1, -1, -1):
                i1 = jnp.where(ls[e] == m1, jnp.full((L,), e, jnp.int32), i1)
            ms = [jnp.where(i1 == jnp.full((L,), e, jnp.int32), neg_inf, ls[e])
                  for e in range(N_EXPERTS)]
            m2 = ms[0]
            for e in range(1, N_EXPERTS):
                m2 = jnp.maximum(m2, ms[e])
            i2 = seven
            for e in range(N_EXPERTS - 1, -1, -1):
                i2 = jnp.where(ms[e] == m2, jnp.full((L,), e, jnp.int32), i2)
            e2 = jnp.exp(m2 - m1)
            denom = 1.0 + e2
            w1 = 1.0 / denom
            w2 = e2 / denom
            for e in range(N_EXPERTS):
                ev = jnp.full((L,), e, jnp.int32)
                ewv[e, sl] = (jnp.where(i1 == ev, w1, zero)
                              + jnp.where(i2 == ev, w2, zero))
        pltpu.sync_copy(ewv, ew_hbm.at[:, pl.ds(base, chunk)])

    return sc_route


@jax.jit
def kernel(x, W, anchors, routing_temp):
    B, T, D = x.shape
    n = B * T
    xf = x.reshape(n, D)
    rtc = jnp.maximum(routing_temp, 0.1).reshape(1, 1)
    wt = W.T
    grid = (n // TILE,)
    q, lt = pl.pallas_call(
        _proj_body,
        grid=grid,
        in_specs=[
            pl.BlockSpec((TILE, D), lambda i: (i, 0)),
            pl.BlockSpec((D, 6), lambda i: (0, 0)),
            pl.BlockSpec((N_EXPERTS, 6), lambda i: (0, 0)),
            pl.BlockSpec((1, 1), lambda i: (0, 0)),
        ],
        out_specs=[
            pl.BlockSpec((6, TILE), lambda i: (0, i)),
            pl.BlockSpec((N_EXPERTS, TILE), lambda i: (0, i)),
        ],
        out_shape=[
            jax.ShapeDtypeStruct((6, n), jnp.float32),
            jax.ShapeDtypeStruct((N_EXPERTS, n), jnp.float32),
        ],
        compiler_params=pltpu.CompilerParams(
            dimension_semantics=("parallel",)),
    )(xf, wt, anchors, rtc)
    ew = _make_sc_routing(n)(lt)
    return ew.T.reshape(B, T, N_EXPERTS), q.T.reshape(B, T, 6)


# speedup vs baseline: 1.3209x; 1.3209x over previous
import jax
import jax.numpy as jnp
from jax.experimental import pallas as pl
from jax.experimental.pallas import tpu as pltpu

QUANT_TEMP = 0.3
TILE = 2048
N_EXPERTS = 7


def _fused_body(x_ref, wt_ref, a_ref, rtc_ref, q_ref, ew_ref):
    xt = x_ref[...]
    z = jax.lax.dot_general(
        xt, wt_ref[...], (((1,), (0,)), ((), ())),
        preferred_element_type=jnp.float32)
    zt = z.T
    qt = jnp.tanh(zt / QUANT_TEMP)
    q_ref[...] = qt
    dott = jax.lax.dot_general(
        a_ref[...], qt, (((1,), (0,)), ((), ())),
        preferred_element_type=jnp.float32)
    hamming = (6.0 - dott) / 2.0
    l = -hamming / rtc_ref[...]
    iota = jax.lax.broadcasted_iota(jnp.int32, l.shape, 0)
    m1 = jnp.max(l, axis=0, keepdims=True)
    i1 = jnp.min(jnp.where(l == m1, iota, N_EXPERTS), axis=0, keepdims=True)
    masked = jnp.where(iota == i1, -jnp.inf, l)
    m2 = jnp.max(masked, axis=0, keepdims=True)
    i2 = jnp.min(jnp.where(masked == m2, iota, N_EXPERTS), axis=0,
                 keepdims=True)
    e2 = jnp.exp(m2 - m1)
    denom = 1.0 + e2
    ew_ref[...] = (jnp.where(iota == i1, 1.0 / denom, 0.0)
                   + jnp.where(iota == i2, e2 / denom, 0.0))


@jax.jit
def kernel(x, W, anchors, routing_temp):
    B, T, D = x.shape
    n = B * T
    xf = x.reshape(n, D)
    rtc = jnp.maximum(routing_temp, 0.1).reshape(1, 1)
    wt = W.T
    grid = (n // TILE,)
    q, ew = pl.pallas_call(
        _fused_body,
        grid=grid,
        in_specs=[
            pl.BlockSpec((TILE, D), lambda i: (i, 0)),
            pl.BlockSpec((D, 6), lambda i: (0, 0)),
            pl.BlockSpec((N_EXPERTS, 6), lambda i: (0, 0)),
            pl.BlockSpec((1, 1), lambda i: (0, 0)),
        ],
        out_specs=[
            pl.BlockSpec((6, TILE), lambda i: (0, i)),
            pl.BlockSpec((N_EXPERTS, TILE), lambda i: (0, i)),
        ],
        out_shape=[
            jax.ShapeDtypeStruct((6, n), jnp.float32),
            jax.ShapeDtypeStruct((N_EXPERTS, n), jnp.float32),
        ],
        compiler_params=pltpu.CompilerParams(
            dimension_semantics=("parallel",)),
    )(xf, wt, anchors, rtc)
    return ew.T.reshape(B, T, N_EXPERTS), q.T.reshape(B, T, 6)
